# baseline (speedup 1.0000x reference)
import jax
import jax.numpy as jnp
from jax.experimental import pallas as pl

_C = 192
_N = 196
_K = 9
_IMB = 2
_EMB = 16

_I0 = (0, 1, 4, 6, 8, 11, 13, 15, 17, 20, 22, 24, 27, 28)


def _resize_kernel(x_ref, rt_ref, o_ref):
    o_ref[...] = jnp.dot(x_ref[...], rt_ref[...],
                         preferred_element_type=jnp.float32)


def _embed_kernel(y_ref, wf_ref, bp_ref, o_ref):
    rows = y_ref[...].reshape(_EMB * 14, _C)
    h = jnp.dot(rows, wf_ref[0], preferred_element_type=jnp.float32)
    o_ref[...] = (h + bp_ref[...]).reshape(_EMB, 1, 14, _C)


def _graph_kernel(h_ref, wg1_ref, wg2_ref, bg_ref,
                  w1_ref, b1_ref, w2_ref, b2_ref, out_ref):
    h_all = h_ref[...].reshape(_IMB * _N, _C)
    iota = jax.lax.broadcasted_iota(jnp.int32, (_N, _N), 1)
    rows = []
    for m in range(_IMB):
        hm = h_all[m * _N:(m + 1) * _N, :]
        sq = jnp.sum(hm * hm, axis=1, keepdims=True)
        gram = jnp.dot(hm, hm.T, preferred_element_type=jnp.float32)
        d = jnp.transpose(sq) - 2.0 * gram
        gmax = jnp.full((_N, _C), -jnp.inf, jnp.float32)
        for _ in range(_K):
            mn = jnp.min(d, axis=1, keepdims=True)
            eq = d == mn
            idx = jnp.min(jnp.where(eq, iota, jnp.int32(2 ** 30)),
                          axis=1, keepdims=True)
            oh = (iota == idx).astype(jnp.float32)
            row = jnp.dot(oh, hm, preferred_element_type=jnp.float32)
            gmax = jnp.maximum(gmax, row)
            d = jnp.where(oh > 0.0, jnp.inf, d)
        mx = gmax - hm
        g = (jnp.dot(hm, wg1_ref[...], preferred_element_type=jnp.float32)
             + jnp.dot(mx, wg2_ref[...], preferred_element_type=jnp.float32)
             + bg_ref[...])
        h2 = hm + g
        f1 = jax.nn.gelu(jnp.dot(h2, w1_ref[...],
                                 preferred_element_type=jnp.float32)
                         + b1_ref[...])
        f = (jnp.dot(f1, w2_ref[...], preferred_element_type=jnp.float32)
             + b2_ref[...])
        h3 = h2 + f
        rows.append(jnp.mean(h3, axis=0, keepdims=True))
    out_ref[...] = jnp.concatenate(rows, axis=0)[None]


def _head_kernel(z_ref, wh1_ref, bh1_ref, wh2_ref, bh2_ref, out_ref):
    t = jax.nn.gelu(jnp.dot(z_ref[...], wh1_ref[...],
                            preferred_element_type=jnp.float32)
                    + bh1_ref[...])
    out_ref[...] = (jnp.dot(t, wh2_ref[...],
                            preferred_element_type=jnp.float32)
                    + bh2_ref[...])


def kernel(x, Wp, bp, Wg, bg, W1, b1, W2, b2, Wh1, bh1, Wh2, bh2):
    b = x.shape[0]
    r = jax.image.resize(jnp.eye(32, dtype=jnp.float32), (224, 32),
                         method='bilinear')

    y = pl.pallas_call(
        _resize_kernel,
        in_specs=[
            pl.BlockSpec((b * 96, 32), lambda: (0, 0)),
            pl.BlockSpec((32, 224), lambda: (0, 0)),
        ],
        out_specs=pl.BlockSpec((b * 96, 224), lambda: (0, 0)),
        out_shape=jax.ShapeDtypeStruct((b * 96, 224), jnp.float32),
    )(x.reshape(b * 96, 32), r.T)

    y5 = y.reshape(b, 3, 32, 14, 16).transpose(0, 3, 1, 2, 4)
    ywin = jnp.stack([y5[:, :, :, i0:i0 + 4, :] for i0 in _I0], axis=1)
    ywin = ywin.reshape(b, 14, 14, _C)

    wp4 = Wp.reshape(3, 16, 16, _C)
    wf = jnp.stack([
        jnp.einsum('ui,cuvk->civk', r[16 * py:16 * py + 16, i0:i0 + 4],
                   wp4).reshape(_C, _C)
        for py, i0 in enumerate(_I0)])

    h4 = pl.pallas_call(
        _embed_kernel,
        grid=(b // _EMB, 14),
        in_specs=[
            pl.BlockSpec((_EMB, 1, 14, _C), lambda i, j: (i, j, 0, 0)),
            pl.BlockSpec((1, _C, _C), lambda i, j: (j, 0, 0)),
            pl.BlockSpec((1, _C), lambda i, j: (0, 0)),
        ],
        out_specs=pl.BlockSpec((_EMB, 1, 14, _C), lambda i, j: (i, j, 0, 0)),
        out_shape=jax.ShapeDtypeStruct((b, 14, 14, _C), jnp.float32),
    )(ywin, wf, bp.reshape(1, _C))
    h = h4.reshape(b, _N, _C)

    wspec = lambda *s: pl.BlockSpec(s, lambda i: (0,) * len(s))
    pooled = pl.pallas_call(
        _graph_kernel,
        grid=(b // _IMB,),
        in_specs=[
            pl.BlockSpec((_IMB, _N, _C), lambda i: (i, 0, 0)),
            wspec(_C, _C),
            wspec(_C, _C),
            wspec(1, _C),
            wspec(_C, 4 * _C),
            wspec(1, 4 * _C),
            wspec(4 * _C, _C),
            wspec(1, _C),
        ],
        out_specs=pl.BlockSpec((1, _IMB, _C), lambda i: (i, 0, 0)),
        out_shape=jax.ShapeDtypeStruct((b // _IMB, _IMB, _C), jnp.float32),
    )(h, Wg[:_C], Wg[_C:], bg.reshape(1, _C),
      W1, b1.reshape(1, 4 * _C), W2, b2.reshape(1, _C))
    pooled = pooled.reshape(b, _C)

    out = pl.pallas_call(
        _head_kernel,
        in_specs=[
            pl.BlockSpec((b, _C), lambda: (0, 0)),
            pl.BlockSpec((_C, 1024), lambda: (0, 0)),
            pl.BlockSpec((1, 1024), lambda: (0, 0)),
            pl.BlockSpec((1024, 10), lambda: (0, 0)),
            pl.BlockSpec((1, 10), lambda: (0, 0)),
        ],
        out_specs=pl.BlockSpec((b, 10), lambda: (0, 0)),
        out_shape=jax.ShapeDtypeStruct((b, 10), jnp.float32),
    )(pooled, Wh1, bh1.reshape(1, 1024), Wh2, bh2.reshape(1, 10))
    return out


# speedup vs baseline: 1.6539x; 1.6539x over previous
import functools

import jax
import jax.numpy as jnp
from jax.experimental import pallas as pl
from jax.experimental.pallas import tpu as pltpu

_C = 192
_N = 196
_K = 9
_IMB = 2


def _resize_kernel(x_ref, rt_ref, o_ref):
    o_ref[...] = jnp.dot(x_ref[...], rt_ref[...],
                         preferred_element_type=jnp.float32
                         ).astype(o_ref.dtype)


def _matmul_resize(x2d, rt, rows_per_step, out_dtype=jnp.float32):
    m, _ = x2d.shape
    grid = m // rows_per_step
    return pl.pallas_call(
        _resize_kernel,
        grid=(grid,),
        in_specs=[
            pl.BlockSpec((rows_per_step, 32), lambda i: (i, 0)),
            pl.BlockSpec((32, 224), lambda i: (0, 0)),
        ],
        out_specs=pl.BlockSpec((rows_per_step, 224), lambda i: (i, 0)),
        out_shape=jax.ShapeDtypeStruct((m, 224), out_dtype),
    )(x2d, rt)


def _graph_kernel(p_ref, wp_ref, bp_ref, wg1_ref, wg2_ref, bg_ref,
                  w1_ref, b1_ref, w2_ref, b2_ref, out_ref):
    pm = p_ref[...].reshape(_IMB * _N, 768)
    h_all = (jnp.dot(pm, wp_ref[...], preferred_element_type=jnp.float32)
             + bp_ref[...])
    iota = jax.lax.broadcasted_iota(jnp.int32, (_N, _N), 1)
    rows = []
    for m in range(_IMB):
        hm = h_all[m * _N:(m + 1) * _N, :]
        sq = jnp.sum(hm * hm, axis=1, keepdims=True)
        gram = jnp.dot(hm, hm.T, preferred_element_type=jnp.float32)
        d = jnp.transpose(sq) - 2.0 * gram
        gmax = jnp.full((_N, _C), -jnp.inf, jnp.float32)
        for _ in range(_K):
            mn = jnp.min(d, axis=1, keepdims=True)
            eq = d == mn
            idx = jnp.min(jnp.where(eq, iota, jnp.int32(2 ** 30)),
                          axis=1, keepdims=True)
            oh = (iota == idx).astype(jnp.float32)
            row = jnp.dot(oh, hm, preferred_element_type=jnp.float32)
            gmax = jnp.maximum(gmax, row)
            d = jnp.where(oh > 0.0, jnp.inf, d)
        mx = gmax - hm
        g = (jnp.dot(hm, wg1_ref[...], preferred_element_type=jnp.float32)
             + jnp.dot(mx, wg2_ref[...], preferred_element_type=jnp.float32)
             + bg_ref[...])
        h2 = hm + g
        f1 = jax.nn.gelu(jnp.dot(h2, w1_ref[...],
                                 preferred_element_type=jnp.float32)
                         + b1_ref[...])
        f = (jnp.dot(f1, w2_ref[...], preferred_element_type=jnp.float32)
             + b2_ref[...])
        h3 = h2 + f
        rows.append(jnp.mean(h3, axis=0, keepdims=True))
    out_ref[...] = jnp.concatenate(rows, axis=0)[None]


def _head_kernel(z_ref, wh1_ref, bh1_ref, wh2_ref, bh2_ref, out_ref):
    t = jax.nn.gelu(jnp.dot(z_ref[...], wh1_ref[...],
                            preferred_element_type=jnp.float32)
                    + bh1_ref[...])
    out_ref[...] = (jnp.dot(t, wh2_ref[...],
                            preferred_element_type=jnp.float32)
                    + bh2_ref[...])


def kernel(x, Wp, bp, Wg, bg, W1, b1, W2, b2, Wh1, bh1, Wh2, bh2):
    b = x.shape[0]
    r = jax.image.resize(jnp.eye(32, dtype=jnp.float32), (224, 32),
                         method='bilinear')
    rt = r.T

    y = _matmul_resize(x.reshape(b * 96, 32), rt, rows_per_step=b * 96)
    yt = y.reshape(b, 3, 32, 224).transpose(0, 3, 1, 2).reshape(b * 672, 32)
    u = _matmul_resize(yt, rt, rows_per_step=4096, out_dtype=jnp.bfloat16)
    p = (u.reshape(b, 14, 16, 3, 14, 16)
         .transpose(0, 4, 1, 3, 5, 2)
         .reshape(b, _N, 768))

    wspec = lambda *s: pl.BlockSpec(s, lambda i: (0,) * len(s))
    pooled = pl.pallas_call(
        _graph_kernel,
        grid=(b // _IMB,),
        in_specs=[
            pl.BlockSpec((_IMB, _N, 768), lambda i: (i, 0, 0)),
            wspec(768, _C),
            wspec(1, _C),
            wspec(_C, _C),
            wspec(_C, _C),
            wspec(1, _C),
            wspec(_C, 4 * _C),
            wspec(1, 4 * _C),
            wspec(4 * _C, _C),
            wspec(1, _C),
        ],
        out_specs=pl.BlockSpec((1, _IMB, _C), lambda i: (i, 0, 0)),
        out_shape=jax.ShapeDtypeStruct((b // _IMB, _IMB, _C), jnp.float32),
    )(p, Wp.astype(jnp.bfloat16), bp.reshape(1, _C),
      Wg[:_C], Wg[_C:], bg.reshape(1, _C),
      W1, b1.reshape(1, 4 * _C), W2, b2.reshape(1, _C))
    pooled = pooled.reshape(b, _C)

    out = pl.pallas_call(
        _head_kernel,
        in_specs=[
            pl.BlockSpec((b, _C), lambda: (0, 0)),
            pl.BlockSpec((_C, 1024), lambda: (0, 0)),
            pl.BlockSpec((1, 1024), lambda: (0, 0)),
            pl.BlockSpec((1024, 10), lambda: (0, 0)),
            pl.BlockSpec((1, 10), lambda: (0, 0)),
        ],
        out_specs=pl.BlockSpec((b, 10), lambda: (0, 0)),
        out_shape=jax.ShapeDtypeStruct((b, 10), jnp.float32),
    )(pooled, Wh1, bh1.reshape(1, 1024), Wh2, bh2.reshape(1, 10))
    return out


# speedup vs baseline: 1.7791x; 1.0757x over previous
import functools

import jax
import jax.numpy as jnp
from jax.experimental import pallas as pl
from jax.experimental.pallas import tpu as pltpu

_C = 192
_N = 196
_K = 9
_IMB = 2


def _resize_kernel(x_ref, rt_ref, o_ref):
    o_ref[...] = jnp.dot(x_ref[...], rt_ref[...].astype(x_ref.dtype),
                         preferred_element_type=jnp.float32
                         ).astype(o_ref.dtype)


def _matmul_resize(x2d, rt, rows_per_step, out_dtype=jnp.float32):
    m, _ = x2d.shape
    grid = m // rows_per_step
    return pl.pallas_call(
        _resize_kernel,
        grid=(grid,),
        in_specs=[
            pl.BlockSpec((rows_per_step, 32), lambda i: (i, 0)),
            pl.BlockSpec((32, 224), lambda i: (0, 0)),
        ],
        out_specs=pl.BlockSpec((rows_per_step, 224), lambda i: (i, 0)),
        out_shape=jax.ShapeDtypeStruct((m, 224), out_dtype),
    )(x2d, rt)


def _graph_kernel(p_ref, wp_ref, bp_ref, wg1_ref, wg2_ref, bg_ref,
                  w1_ref, b1_ref, w2_ref, b2_ref, out_ref):
    pm = p_ref[...].reshape(_IMB * _N, 768)
    h_all = (jnp.dot(pm, wp_ref[...], preferred_element_type=jnp.float32)
             + bp_ref[...])
    iota = jax.lax.broadcasted_iota(jnp.int32, (_N, _N), 1)
    rows = []
    for m in range(_IMB):
        hm = h_all[m * _N:(m + 1) * _N, :]
        hm_bf = hm.astype(jnp.bfloat16)
        sq = jnp.sum(hm * hm, axis=1, keepdims=True)
        gram = jnp.dot(hm, hm.T, preferred_element_type=jnp.float32)
        d = jnp.transpose(sq) - 2.0 * gram
        gmax = jnp.full((_N, _C), -jnp.inf, jnp.float32)
        for _ in range(_K):
            amin = jnp.argmin(d, axis=1)[:, None]
            oh = (iota == amin)
            row = jnp.dot(oh.astype(jnp.bfloat16), hm_bf,
                          preferred_element_type=jnp.float32)
            gmax = jnp.maximum(gmax, row)
            d = jnp.where(oh, jnp.inf, d)
        mx = gmax - hm
        g = (jnp.dot(hm, wg1_ref[...], preferred_element_type=jnp.float32)
             + jnp.dot(mx, wg2_ref[...], preferred_element_type=jnp.float32)
             + bg_ref[...])
        h2 = hm + g
        f1 = jax.nn.gelu(jnp.dot(h2, w1_ref[...],
                                 preferred_element_type=jnp.float32)
                         + b1_ref[...])
        f = (jnp.dot(f1, w2_ref[...], preferred_element_type=jnp.float32)
             + b2_ref[...])
        h3 = h2 + f
        rows.append(jnp.mean(h3, axis=0, keepdims=True))
    out_ref[...] = jnp.concatenate(rows, axis=0)[None]


def _head_kernel(z_ref, wh1_ref, bh1_ref, wh2_ref, bh2_ref, out_ref):
    t = jax.nn.gelu(jnp.dot(z_ref[...], wh1_ref[...],
                            preferred_element_type=jnp.float32)
                    + bh1_ref[...])
    out_ref[...] = (jnp.dot(t, wh2_ref[...],
                            preferred_element_type=jnp.float32)
                    + bh2_ref[...])


def kernel(x, Wp, bp, Wg, bg, W1, b1, W2, b2, Wh1, bh1, Wh2, bh2):
    b = x.shape[0]
    r = jax.image.resize(jnp.eye(32, dtype=jnp.float32), (224, 32),
                         method='bilinear')
    rt = r.T

    y = _matmul_resize(x.reshape(b * 96, 32), rt, rows_per_step=b * 96,
                       out_dtype=jnp.bfloat16)
    yt = y.reshape(b, 3, 32, 224).transpose(0, 3, 1, 2).reshape(b * 672, 32)
    u = _matmul_resize(yt, rt, rows_per_step=4096, out_dtype=jnp.bfloat16)
    p = (u.reshape(b, 14, 16, 3, 14, 16)
         .transpose(0, 4, 1, 3, 5, 2)
         .reshape(b, _N, 768))

    wspec = lambda *s: pl.BlockSpec(s, lambda i: (0,) * len(s))
    pooled = pl.pallas_call(
        _graph_kernel,
        grid=(b // _IMB,),
        in_specs=[
            pl.BlockSpec((_IMB, _N, 768), lambda i: (i, 0, 0)),
            wspec(768, _C),
            wspec(1, _C),
            wspec(_C, _C),
            wspec(_C, _C),
            wspec(1, _C),
            wspec(_C, 4 * _C),
            wspec(1, 4 * _C),
            wspec(4 * _C, _C),
            wspec(1, _C),
        ],
        out_specs=pl.BlockSpec((1, _IMB, _C), lambda i: (i, 0, 0)),
        out_shape=jax.ShapeDtypeStruct((b // _IMB, _IMB, _C), jnp.float32),
    )(p, Wp.astype(jnp.bfloat16), bp.reshape(1, _C),
      Wg[:_C], Wg[_C:], bg.reshape(1, _C),
      W1, b1.reshape(1, 4 * _C), W2, b2.reshape(1, _C))
    pooled = pooled.reshape(b, _C)

    out = pl.pallas_call(
        _head_kernel,
        in_specs=[
            pl.BlockSpec((b, _C), lambda: (0, 0)),
            pl.BlockSpec((_C, 1024), lambda: (0, 0)),
            pl.BlockSpec((1, 1024), lambda: (0, 0)),
            pl.BlockSpec((1024, 10), lambda: (0, 0)),
            pl.BlockSpec((1, 10), lambda: (0, 0)),
        ],
        out_specs=pl.BlockSpec((b, 10), lambda: (0, 0)),
        out_shape=jax.ShapeDtypeStruct((b, 10), jnp.float32),
    )(pooled, Wh1, bh1.reshape(1, 1024), Wh2, bh2.reshape(1, 10))
    return out


# speedup vs baseline: 2.0045x; 1.1267x over previous
import functools

import jax
import jax.numpy as jnp
from jax.experimental import pallas as pl
from jax.experimental.pallas import tpu as pltpu

_C = 192
_N = 196
_K = 9
_IMB = 2


def _resize_kernel(x_ref, rt_ref, o_ref):
    o_ref[...] = jnp.dot(x_ref[...], rt_ref[...].astype(x_ref.dtype),
                         preferred_element_type=jnp.float32
                         ).astype(o_ref.dtype)


def _matmul_resize(x2d, rt, rows_per_step, out_dtype=jnp.float32):
    m, _ = x2d.shape
    grid = m // rows_per_step
    return pl.pallas_call(
        _resize_kernel,
        grid=(grid,),
        in_specs=[
            pl.BlockSpec((rows_per_step, 32), lambda i: (i, 0)),
            pl.BlockSpec((32, 224), lambda i: (0, 0)),
        ],
        out_specs=pl.BlockSpec((rows_per_step, 224), lambda i: (i, 0)),
        out_shape=jax.ShapeDtypeStruct((m, 224), out_dtype),
    )(x2d, rt)


def _graph_kernel(p_ref, wp_ref, bp_ref, wg1_ref, wg2_ref, bg_ref,
                  w1_ref, b1_ref, w2_ref, b2_ref, out_ref):
    pm = p_ref[...].reshape(_IMB * _N, 768)
    h_all = (jnp.dot(pm, wp_ref[...], preferred_element_type=jnp.float32)
             + bp_ref[...])
    iota0 = jax.lax.broadcasted_iota(jnp.int32, (_N, _N), 0)
    rows = []
    for m in range(_IMB):
        hm = h_all[m * _N:(m + 1) * _N, :]
        hm_bf = hm.astype(jnp.bfloat16)
        sq = jnp.sum(hm * hm, axis=1, keepdims=True)
        gram = jnp.dot(hm, hm.T, preferred_element_type=jnp.float32)
        d = sq - 2.0 * gram
        gmax = jnp.full((_N, _C), -jnp.inf, jnp.float32)
        for _ in range(_K):
            amin = jnp.argmin(d, axis=0)[None, :]
            oh = (iota0 == amin)
            row = jax.lax.dot_general(
                oh.astype(jnp.bfloat16), hm_bf,
                (((0,), (0,)), ((), ())),
                preferred_element_type=jnp.float32)
            gmax = jnp.maximum(gmax, row)
            d = jnp.where(oh, jnp.inf, d)
        mx_bf = (gmax - hm).astype(jnp.bfloat16)
        g = (jnp.dot(hm_bf, wg1_ref[...], preferred_element_type=jnp.float32)
             + jnp.dot(mx_bf, wg2_ref[...], preferred_element_type=jnp.float32)
             + bg_ref[...])
        h2 = hm + g
        f1 = jax.nn.gelu(jnp.dot(h2.astype(jnp.bfloat16), w1_ref[...],
                                 preferred_element_type=jnp.float32)
                         + b1_ref[...])
        f = (jnp.dot(f1.astype(jnp.bfloat16), w2_ref[...],
                     preferred_element_type=jnp.float32)
             + b2_ref[...])
        h3 = h2 + f
        rows.append(jnp.mean(h3, axis=0, keepdims=True))
    out_ref[...] = jnp.concatenate(rows, axis=0)[None]


def _head_kernel(z_ref, wh1_ref, bh1_ref, wh2_ref, bh2_ref, out_ref):
    t = jax.nn.gelu(jnp.dot(z_ref[...], wh1_ref[...],
                            preferred_element_type=jnp.float32)
                    + bh1_ref[...])
    out_ref[...] = (jnp.dot(t, wh2_ref[...],
                            preferred_element_type=jnp.float32)
                    + bh2_ref[...])


def kernel(x, Wp, bp, Wg, bg, W1, b1, W2, b2, Wh1, bh1, Wh2, bh2):
    b = x.shape[0]
    r = jax.image.resize(jnp.eye(32, dtype=jnp.float32), (224, 32),
                         method='bilinear')
    rt = r.T

    y = _matmul_resize(x.reshape(b * 96, 32), rt, rows_per_step=b * 96,
                       out_dtype=jnp.bfloat16)
    yt = y.reshape(b, 3, 32, 224).transpose(0, 3, 1, 2).reshape(b * 672, 32)
    u = _matmul_resize(yt, rt, rows_per_step=4096, out_dtype=jnp.bfloat16)
    p = (u.reshape(b, 14, 16, 3, 14, 16)
         .transpose(0, 4, 1, 3, 5, 2)
         .reshape(b, _N, 768))

    wspec = lambda *s: pl.BlockSpec(s, lambda i: (0,) * len(s))
    pooled = pl.pallas_call(
        _graph_kernel,
        grid=(b // _IMB,),
        in_specs=[
            pl.BlockSpec((_IMB, _N, 768), lambda i: (i, 0, 0)),
            wspec(768, _C),
            wspec(1, _C),
            wspec(_C, _C),
            wspec(_C, _C),
            wspec(1, _C),
            wspec(_C, 4 * _C),
            wspec(1, 4 * _C),
            wspec(4 * _C, _C),
            wspec(1, _C),
        ],
        out_specs=pl.BlockSpec((1, _IMB, _C), lambda i: (i, 0, 0)),
        out_shape=jax.ShapeDtypeStruct((b // _IMB, _IMB, _C), jnp.float32),
    )(p, Wp.astype(jnp.bfloat16), bp.reshape(1, _C),
      Wg[:_C].astype(jnp.bfloat16), Wg[_C:].astype(jnp.bfloat16),
      bg.reshape(1, _C),
      W1.astype(jnp.bfloat16), b1.reshape(1, 4 * _C),
      W2.astype(jnp.bfloat16), b2.reshape(1, _C))
    pooled = pooled.reshape(b, _C)

    out = pl.pallas_call(
        _head_kernel,
        in_specs=[
            pl.BlockSpec((b, _C), lambda: (0, 0)),
            pl.BlockSpec((_C, 1024), lambda: (0, 0)),
            pl.BlockSpec((1, 1024), lambda: (0, 0)),
            pl.BlockSpec((1024, 10), lambda: (0, 0)),
            pl.BlockSpec((1, 10), lambda: (0, 0)),
        ],
        out_specs=pl.BlockSpec((b, 10), lambda: (0, 0)),
        out_shape=jax.ShapeDtypeStruct((b, 10), jnp.float32),
    )(pooled, Wh1, bh1.reshape(1, 1024), Wh2, bh2.reshape(1, 10))
    return out


# speedup vs baseline: 4.7392x; 2.3643x over previous
import jax
import jax.numpy as jnp
from jax.experimental import pallas as pl

_C = 192
_N = 196
_K = 9
_IMB = 2
_EMB = 16


def _resize_kernel(x_ref, rt_ref, o_ref):
    o_ref[...] = jnp.dot(x_ref[...], rt_ref[...],
                         preferred_element_type=jnp.float32
                         ).astype(o_ref.dtype)


def _embed_kernel(y_ref, wf_ref, bp_ref, o_ref):
    yt4 = y_ref[...].reshape(_EMB, 16, 14, 96)
    ycat = jnp.concatenate([yt4[:, v] for v in range(16)],
                           axis=2).reshape(_EMB * 14, 1536)
    outs = []
    for py in range(14):
        hp = (jnp.dot(ycat, wf_ref[py], preferred_element_type=jnp.float32)
              + bp_ref[...])
        outs.append(hp.reshape(_EMB, 1, 14, _C))
    o_ref[...] = jnp.concatenate(outs, axis=1)


def _graph_kernel(h_ref, wg1_ref, wg2_ref, bg_ref,
                  w1_ref, b1_ref, w2_ref, b2_ref, out_ref):
    h_all = h_ref[...].reshape(_IMB * _N, _C)
    iota0 = jax.lax.broadcasted_iota(jnp.int32, (_N, _N), 0)
    rows = []
    for m in range(_IMB):
        hm = h_all[m * _N:(m + 1) * _N, :]
        hm_bf = hm.astype(jnp.bfloat16)
        sq = jnp.sum(hm * hm, axis=1, keepdims=True)
        gram = jnp.dot(hm, hm.T, preferred_element_type=jnp.float32)
        d = sq - 2.0 * gram
        gmax = jnp.full((_N, _C), -jnp.inf, jnp.float32)
        for _ in range(_K):
            amin = jnp.argmin(d, axis=0)[None, :]
            oh = (iota0 == amin)
            row = jax.lax.dot_general(
                oh.astype(jnp.bfloat16), hm_bf,
                (((0,), (0,)), ((), ())),
                preferred_element_type=jnp.float32)
            gmax = jnp.maximum(gmax, row)
            d = jnp.where(oh, jnp.inf, d)
        mx_bf = (gmax - hm).astype(jnp.bfloat16)
        g = (jnp.dot(hm_bf, wg1_ref[...], preferred_element_type=jnp.float32)
             + jnp.dot(mx_bf, wg2_ref[...], preferred_element_type=jnp.float32)
             + bg_ref[...])
        h2 = hm + g
        f1 = jax.nn.gelu(jnp.dot(h2.astype(jnp.bfloat16), w1_ref[...],
                                 preferred_element_type=jnp.float32)
                         + b1_ref[...])
        f = (jnp.dot(f1.astype(jnp.bfloat16), w2_ref[...],
                     preferred_element_type=jnp.float32)
             + b2_ref[...])
        h3 = h2 + f
        rows.append(jnp.mean(h3, axis=0, keepdims=True))
    out_ref[...] = jnp.concatenate(rows, axis=0)[None]


def _head_kernel(z_ref, wh1_ref, bh1_ref, wh2_ref, bh2_ref, out_ref):
    t = jax.nn.gelu(jnp.dot(z_ref[...], wh1_ref[...],
                            preferred_element_type=jnp.float32)
                    + bh1_ref[...])
    out_ref[...] = (jnp.dot(t, wh2_ref[...],
                            preferred_element_type=jnp.float32)
                    + bh2_ref[...])


def kernel(x, Wp, bp, Wg, bg, W1, b1, W2, b2, Wh1, bh1, Wh2, bh2):
    b = x.shape[0]
    r = jax.image.resize(jnp.eye(32, dtype=jnp.float32), (224, 32),
                         method='bilinear')
    perm = jnp.arange(224).reshape(14, 16).T.reshape(224)
    rt_p = r.T[:, perm]

    y = pl.pallas_call(
        _resize_kernel,
        in_specs=[
            pl.BlockSpec((b * 96, 32), lambda: (0, 0)),
            pl.BlockSpec((32, 224), lambda: (0, 0)),
        ],
        out_specs=pl.BlockSpec((b * 96, 224), lambda: (0, 0)),
        out_shape=jax.ShapeDtypeStruct((b * 96, 224), jnp.float32),
    )(x.reshape(b * 96, 32), rt_p)

    yt = (y.reshape(b, 3, 32, 224).transpose(0, 3, 1, 2)
          .reshape(b, 224, 96))

    wp4 = Wp.reshape(3, 16, 16, _C)
    wf = jnp.stack([
        jnp.einsum('ui,cuvk->vcik', r[16 * py:16 * py + 16, :], wp4)
        .reshape(16 * 96, _C)
        for py in range(14)])

    h4 = pl.pallas_call(
        _embed_kernel,
        grid=(b // _EMB,),
        in_specs=[
            pl.BlockSpec((_EMB, 224, 96), lambda i: (i, 0, 0)),
            pl.BlockSpec((14, 1536, _C), lambda i: (0, 0, 0)),
            pl.BlockSpec((1, _C), lambda i: (0, 0)),
        ],
        out_specs=pl.BlockSpec((_EMB, 14, 14, _C), lambda i: (i, 0, 0, 0)),
        out_shape=jax.ShapeDtypeStruct((b, 14, 14, _C), jnp.float32),
    )(yt, wf, bp.reshape(1, _C))
    h = h4.reshape(b, _N, _C)

    wspec = lambda *s: pl.BlockSpec(s, lambda i: (0,) * len(s))
    pooled = pl.pallas_call(
        _graph_kernel,
        grid=(b // _IMB,),
        in_specs=[
            pl.BlockSpec((_IMB, _N, _C), lambda i: (i, 0, 0)),
            wspec(_C, _C),
            wspec(_C, _C),
            wspec(1, _C),
            wspec(_C, 4 * _C),
            wspec(1, 4 * _C),
            wspec(4 * _C, _C),
            wspec(1, _C),
        ],
        out_specs=pl.BlockSpec((1, _IMB, _C), lambda i: (i, 0, 0)),
        out_shape=jax.ShapeDtypeStruct((b // _IMB, _IMB, _C), jnp.float32),
    )(h, Wg[:_C].astype(jnp.bfloat16), Wg[_C:].astype(jnp.bfloat16),
      bg.reshape(1, _C),
      W1.astype(jnp.bfloat16), b1.reshape(1, 4 * _C),
      W2.astype(jnp.bfloat16), b2.reshape(1, _C))
    pooled = pooled.reshape(b, _C)

    out = pl.pallas_call(
        _head_kernel,
        in_specs=[
            pl.BlockSpec((b, _C), lambda: (0, 0)),
            pl.BlockSpec((_C, 1024), lambda: (0, 0)),
            pl.BlockSpec((1, 1024), lambda: (0, 0)),
            pl.BlockSpec((1024, 10), lambda: (0, 0)),
            pl.BlockSpec((1, 10), lambda: (0, 0)),
        ],
        out_specs=pl.BlockSpec((b, 10), lambda: (0, 0)),
        out_shape=jax.ShapeDtypeStruct((b, 10), jnp.float32),
    )(pooled, Wh1, bh1.reshape(1, 1024), Wh2, bh2.reshape(1, 10))
    return out


# speedup vs baseline: 4.8803x; 1.0298x over previous
import jax
import jax.numpy as jnp
from jax.experimental import pallas as pl

_C = 192
_N = 196
_K = 9
_IMB = 4
_EMB = 16


def _resize_kernel(x_ref, rt_ref, o_ref):
    o_ref[...] = jnp.dot(x_ref[...], rt_ref[...],
                         preferred_element_type=jnp.float32
                         ).astype(o_ref.dtype)


def _embed_kernel(y_ref, wf_ref, bp_ref, o_ref):
    yt4 = y_ref[...].reshape(_EMB, 16, 14, 96)
    ycat = jnp.concatenate([yt4[:, v] for v in range(16)],
                           axis=2).reshape(_EMB * 14, 1536)
    outs = []
    for py in range(14):
        hp = (jnp.dot(ycat, wf_ref[py], preferred_element_type=jnp.float32)
              + bp_ref[...])
        outs.append(hp.reshape(_EMB, 1, 14, _C))
    o_ref[...] = jnp.concatenate(outs, axis=1)


def _graph_kernel(h_ref, wg1_ref, wg2_ref, bg_ref,
                  w1_ref, b1_ref, w2_ref, b2_ref, out_ref):
    h_all = h_ref[...].reshape(_IMB * _N, _C)
    iota0 = jax.lax.broadcasted_iota(jnp.int32, (_N, _N), 0)
    rows = []
    for m in range(_IMB):
        hm = h_all[m * _N:(m + 1) * _N, :]
        hm_bf = hm.astype(jnp.bfloat16)
        sq = jnp.sum(hm * hm, axis=1, keepdims=True)
        gram = jnp.dot(hm, hm.T, preferred_element_type=jnp.float32)
        d = sq - 2.0 * gram
        gmax = jnp.full((_N, _C), -jnp.inf, jnp.float32)
        for _ in range(_K):
            amin = jnp.argmin(d, axis=0)[None, :]
            oh = (iota0 == amin)
            row = jax.lax.dot_general(
                oh.astype(jnp.bfloat16), hm_bf,
                (((0,), (0,)), ((), ())),
                preferred_element_type=jnp.float32)
            gmax = jnp.maximum(gmax, row)
            d = jnp.where(oh, jnp.inf, d)
        mx_bf = (gmax - hm).astype(jnp.bfloat16)
        g = (jnp.dot(hm_bf, wg1_ref[...], preferred_element_type=jnp.float32)
             + jnp.dot(mx_bf, wg2_ref[...], preferred_element_type=jnp.float32)
             + bg_ref[...])
        h2 = hm + g
        f1 = jax.nn.gelu(jnp.dot(h2.astype(jnp.bfloat16), w1_ref[...],
                                 preferred_element_type=jnp.float32)
                         + b1_ref[...])
        f = (jnp.dot(f1.astype(jnp.bfloat16), w2_ref[...],
                     preferred_element_type=jnp.float32)
             + b2_ref[...])
        h3 = h2 + f
        rows.append(jnp.mean(h3, axis=0, keepdims=True))
    out_ref[...] = jnp.concatenate(rows, axis=0)[None]


def _head_kernel(z_ref, wh1_ref, bh1_ref, wh2_ref, bh2_ref, out_ref):
    t = jax.nn.gelu(jnp.dot(z_ref[...], wh1_ref[...],
                            preferred_element_type=jnp.float32)
                    + bh1_ref[...])
    out_ref[...] = (jnp.dot(t, wh2_ref[...],
                            preferred_element_type=jnp.float32)
                    + bh2_ref[...])


def kernel(x, Wp, bp, Wg, bg, W1, b1, W2, b2, Wh1, bh1, Wh2, bh2):
    b = x.shape[0]
    r = jax.image.resize(jnp.eye(32, dtype=jnp.float32), (224, 32),
                         method='bilinear')
    perm = jnp.arange(224).reshape(14, 16).T.reshape(224)
    rt_p = r.T[:, perm]

    y = pl.pallas_call(
        _resize_kernel,
        in_specs=[
            pl.BlockSpec((b * 96, 32), lambda: (0, 0)),
            pl.BlockSpec((32, 224), lambda: (0, 0)),
        ],
        out_specs=pl.BlockSpec((b * 96, 224), lambda: (0, 0)),
        out_shape=jax.ShapeDtypeStruct((b * 96, 224), jnp.float32),
    )(x.reshape(b * 96, 32), rt_p)

    yt = (y.reshape(b, 3, 32, 224).transpose(0, 3, 1, 2)
          .reshape(b, 224, 96))

    wp4 = Wp.reshape(3, 16, 16, _C)
    wf = jnp.stack([
        jnp.einsum('ui,cuvk->vcik', r[16 * py:16 * py + 16, :], wp4)
        .reshape(16 * 96, _C)
        for py in range(14)])

    h4 = pl.pallas_call(
        _embed_kernel,
        grid=(b // _EMB,),
        in_specs=[
            pl.BlockSpec((_EMB, 224, 96), lambda i: (i, 0, 0)),
            pl.BlockSpec((14, 1536, _C), lambda i: (0, 0, 0)),
            pl.BlockSpec((1, _C), lambda i: (0, 0)),
        ],
        out_specs=pl.BlockSpec((_EMB, 14, 14, _C), lambda i: (i, 0, 0, 0)),
        out_shape=jax.ShapeDtypeStruct((b, 14, 14, _C), jnp.float32),
    )(yt, wf, bp.reshape(1, _C))
    h = h4.reshape(b, _N, _C)

    wspec = lambda *s: pl.BlockSpec(s, lambda i: (0,) * len(s))
    pooled = pl.pallas_call(
        _graph_kernel,
        grid=(b // _IMB,),
        in_specs=[
            pl.BlockSpec((_IMB, _N, _C), lambda i: (i, 0, 0)),
            wspec(_C, _C),
            wspec(_C, _C),
            wspec(1, _C),
            wspec(_C, 4 * _C),
            wspec(1, 4 * _C),
            wspec(4 * _C, _C),
            wspec(1, _C),
        ],
        out_specs=pl.BlockSpec((1, _IMB, _C), lambda i: (i, 0, 0)),
        out_shape=jax.ShapeDtypeStruct((b // _IMB, _IMB, _C), jnp.float32),
    )(h, Wg[:_C].astype(jnp.bfloat16), Wg[_C:].astype(jnp.bfloat16),
      bg.reshape(1, _C),
      W1.astype(jnp.bfloat16), b1.reshape(1, 4 * _C),
      W2.astype(jnp.bfloat16), b2.reshape(1, _C))
    pooled = pooled.reshape(b, _C)

    out = pl.pallas_call(
        _head_kernel,
        in_specs=[
            pl.BlockSpec((b, _C), lambda: (0, 0)),
            pl.BlockSpec((_C, 1024), lambda: (0, 0)),
            pl.BlockSpec((1, 1024), lambda: (0, 0)),
            pl.BlockSpec((1024, 10), lambda: (0, 0)),
            pl.BlockSpec((1, 10), lambda: (0, 0)),
        ],
        out_specs=pl.BlockSpec((b, 10), lambda: (0, 0)),
        out_shape=jax.ShapeDtypeStruct((b, 10), jnp.float32),
    )(pooled, Wh1, bh1.reshape(1, 1024), Wh2, bh2.reshape(1, 10))
    return out
